# Initial kernel scaffold; baseline (speedup 1.0000x reference)
import jax
import jax.numpy as jnp
from jax.experimental import pallas as pl


def kernel(x, w_gate, w_noise, W1, b1, W2, b2):
    raise NotImplementedError("write your pallas kernel here")


# speedup vs baseline: 1.4497x; 1.4497x over previous
import functools

import jax
import jax.numpy as jnp
import numpy as np
from jax.experimental import pallas as pl
from jax.experimental.pallas import tpu as pltpu

_N = 2048
_D = 1024
_E = 8
_DFF = 1024
_LANE = 128
_NEG = -3.0e38
_EPS = float(np.finfo(np.float64).eps)


def _gating_body(x_ref, wg_ref, gates_ref, loss_ref):
    x = x_ref[...]
    wg = wg_ref[...]
    logits = jnp.dot(x, wg, preferred_element_type=jnp.float32)
    n = logits.shape[0]
    col = jax.lax.broadcasted_iota(jnp.int32, (n, _LANE), 1)
    valid = col < _E
    neg = jnp.where(valid, logits, _NEG)
    m1 = jnp.max(neg, axis=1, keepdims=True)
    i1 = jnp.min(jnp.where(neg == m1, col, _LANE), axis=1, keepdims=True)
    neg2 = jnp.where(col == i1, _NEG, neg)
    m2 = jnp.max(neg2, axis=1, keepdims=True)
    i2 = jnp.min(jnp.where(neg2 == m2, col, _LANE), axis=1, keepdims=True)
    t = jnp.exp(m2 - m1)
    g1 = 1.0 / (1.0 + t)
    g2 = t / (1.0 + t)
    gates = jnp.where(col == i1, g1, 0.0) + jnp.where(col == i2, g2, 0.0)
    gates = jnp.where(valid, gates, 0.0)
    gates_ref[...] = gates
    imp = jnp.sum(gates, axis=0, keepdims=True)
    load = jnp.sum((gates > 0.0).astype(jnp.float32), axis=0, keepdims=True)

    def _cv2(v):
        mean = jnp.sum(jnp.where(col[:1] < _E, v, 0.0)) / _E
        var = jnp.sum(jnp.where(col[:1] < _E, (v - mean) ** 2, 0.0)) / (_E - 1)
        return var / (mean * mean + 1e-10)

    loss_ref[0, 0] = _cv2(imp) + _cv2(load)


def _moe_body(x_ref, w1_ref, b1_ref, w2_ref, b2_ref, gates_ref, y_ref, acc_ref):
    e = pl.program_id(0)
    nb = pl.program_id(1)
    x = x_ref[...]
    h = jnp.dot(x, w1_ref[0], preferred_element_type=jnp.float32) + b1_ref[0]
    h = jnp.maximum(h, 0.0)
    out = jnp.dot(h, w2_ref[0], preferred_element_type=jnp.float32) + b2_ref[0]
    col = jax.lax.broadcasted_iota(jnp.int32, gates_ref.shape, 1)
    g = jnp.sum(jnp.where(col == e, gates_ref[...], 0.0), axis=1, keepdims=True)
    contrib = g * jnp.exp(out)
    bn = x.shape[0]
    sl = pl.ds(nb * bn, bn)

    @pl.when(e == 0)
    def _():
        acc_ref[sl, :] = contrib

    @pl.when(e > 0)
    def _():
        acc_ref[sl, :] += contrib

    @pl.when(e == _E - 1)
    def _():
        acc = acc_ref[sl, :]
        y_ref[...] = jnp.log(jnp.where(acc == 0.0, _EPS, acc))


def kernel(x, w_gate, w_noise, W1, b1, W2, b2):
    del w_noise
    n, d = x.shape
    wg_pad = jnp.pad(w_gate, ((0, 0), (0, _LANE - _E)))
    gates, loss = pl.pallas_call(
        _gating_body,
        out_shape=(
            jax.ShapeDtypeStruct((n, _LANE), jnp.float32),
            jax.ShapeDtypeStruct((1, 1), jnp.float32),
        ),
        in_specs=[
            pl.BlockSpec((n, d), lambda: (0, 0)),
            pl.BlockSpec((d, _LANE), lambda: (0, 0)),
        ],
        out_specs=(
            pl.BlockSpec((n, _LANE), lambda: (0, 0)),
            pl.BlockSpec(memory_space=pltpu.SMEM),
        ),
    )(x, wg_pad)

    bn = 256
    nblocks = n // bn
    y = pl.pallas_call(
        _moe_body,
        grid=(_E, nblocks),
        out_shape=jax.ShapeDtypeStruct((n, d), jnp.float32),
        in_specs=[
            pl.BlockSpec((bn, d), lambda e, nb: (nb, 0)),
            pl.BlockSpec((1, d, _DFF), lambda e, nb: (e, 0, 0)),
            pl.BlockSpec((1, 1, _DFF), lambda e, nb: (e, 0, 0)),
            pl.BlockSpec((1, _DFF, d), lambda e, nb: (e, 0, 0)),
            pl.BlockSpec((1, 1, d), lambda e, nb: (e, 0, 0)),
            pl.BlockSpec((bn, _LANE), lambda e, nb: (nb, 0)),
        ],
        out_specs=pl.BlockSpec((bn, d), lambda e, nb: (nb, 0)),
        scratch_shapes=[pltpu.VMEM((n, d), jnp.float32)],
    )(x, W1, b1[:, None, :], W2, b2[:, None, :], gates)
    return y, loss[0, 0]
